# baseline (speedup 1.0000x reference)
import functools

import jax
import jax.numpy as jnp
from jax import lax
from jax.experimental import pallas as pl
from jax.experimental.pallas import tpu as pltpu
from jax.experimental.pallas import tpu_sc as plsc

B = 16384
D = 64
DM = 256
NROW = 100000

NC = 2
NS = 16
NW = NC * NS
BPW = B // NW
CH = 128
NCHUNK = BPW // CH
CHG = 256
NCHUNKG = BPW // CHG


def _sc_mesh():
    return plsc.VectorSubcoreMesh(core_axis_name="c", subcore_axis_name="s",
                                  num_cores=NC, num_subcores=NS)


@functools.lru_cache(maxsize=None)
def _get_sc_gather_mlp():
    @functools.partial(
        pl.kernel,
        mesh=_sc_mesh(),
        out_type=[
            jax.ShapeDtypeStruct((B, DM), jnp.float32),
            jax.ShapeDtypeStruct((B, DM), jnp.float32),
        ],
        scratch_types=[
            pltpu.VMEM((CH,), jnp.int32),
            pltpu.VMEM((CH,), jnp.int32),
            pltpu.VMEM((CH, DM), jnp.float32),
            pltpu.VMEM((CH, DM), jnp.float32),
            pltpu.SemaphoreType.DMA,
        ],
    )
    def _sc_gather_mlp(user_hbm, item_hbm, um_t, im_t, um_out, im_out,
                       idxu_v, idxi_v, um_v, im_v, sem):
        wid = lax.axis_index("s") * NC + lax.axis_index("c")
        base = wid * BPW
        for c in range(NCHUNK):
            off = base + c * CH
            pltpu.sync_copy(user_hbm.at[pl.ds(off, CH)], idxu_v)
            pltpu.sync_copy(item_hbm.at[pl.ds(off, CH)], idxi_v)
            cps = [
                pltpu.async_copy(um_t.at[idxu_v], um_v, sem),
                pltpu.async_copy(im_t.at[idxi_v], im_v, sem),
            ]
            for cp in cps:
                cp.wait()
            pltpu.sync_copy(um_v, um_out.at[pl.ds(off, CH)])
            pltpu.sync_copy(im_v, im_out.at[pl.ds(off, CH)])

    return _sc_gather_mlp


@functools.lru_cache(maxsize=None)
def _get_sc_gather_gmf():
    @functools.partial(
        pl.kernel,
        mesh=_sc_mesh(),
        out_type=[
            jax.ShapeDtypeStruct((B, 2 * D), jnp.float32),
            jax.ShapeDtypeStruct((B, 2 * D), jnp.float32),
        ],
        scratch_types=[
            pltpu.VMEM((CHG,), jnp.int32),
            pltpu.VMEM((CHG,), jnp.int32),
            pltpu.VMEM((CHG, 2 * D), jnp.float32),
            pltpu.VMEM((CHG, 2 * D), jnp.float32),
            pltpu.SemaphoreType.DMA,
        ],
    )
    def _sc_gather_gmf(userh_hbm, itemh_hbm, ug_t, ig_t, ug_out, ig_out,
                       idxu_v, idxi_v, ug_v, ig_v, sem):
        wid = lax.axis_index("s") * NC + lax.axis_index("c")
        base = wid * BPW
        for c in range(NCHUNKG):
            off = base + c * CHG
            pltpu.sync_copy(userh_hbm.at[pl.ds(off, CHG)], idxu_v)
            pltpu.sync_copy(itemh_hbm.at[pl.ds(off, CHG)], idxi_v)
            cps = [
                pltpu.async_copy(ug_t.at[idxu_v], ug_v, sem),
                pltpu.async_copy(ig_t.at[idxi_v], ig_v, sem),
            ]
            for cp in cps:
                cp.wait()
            pltpu.sync_copy(ug_v, ug_out.at[pl.ds(off, CHG)])
            pltpu.sync_copy(ig_v, ig_out.at[pl.ds(off, CHG)])

    return _sc_gather_gmf


RPB = 5000
HROW = NROW // 2


def _repack_body(a_ref, o_ref):
    j = pl.program_id(1)

    @pl.when(j == 0)
    def _():
        o_ref[:, :D] = a_ref[...]

    @pl.when(j == 1)
    def _():
        o_ref[:, D:] = a_ref[...]


def _tc_repack(table):
    nblk = HROW // RPB
    return pl.pallas_call(
        _repack_body,
        grid=(nblk, 2),
        in_specs=[pl.BlockSpec((RPB, D), lambda i, j: (i + j * nblk, 0))],
        out_specs=pl.BlockSpec((RPB, 2 * D), lambda i, j: (i, 0)),
        out_shape=jax.ShapeDtypeStruct((HROW, 2 * D), jnp.float32),
    )(table)


BT = 2048


def _mlp_body(u, it, ugr, igr, um, im, w1a, w1b, b1, w2t, b2, w3t, b3,
              wpg, wpm, bp, out):
    h = jnp.dot(um[...], w1a[...], preferred_element_type=jnp.float32)
    h = h + jnp.dot(im[...], w1b[...], preferred_element_type=jnp.float32)
    h = jnp.maximum(h + b1[...], 0.0)
    h = jnp.maximum(jnp.dot(h, w2t[...], preferred_element_type=jnp.float32)
                    + b2[...], 0.0)
    m = jnp.maximum(jnp.dot(h, w3t[...], preferred_element_type=jnp.float32)
                    + b3[...], 0.0)
    ug = jnp.where(u[...] >= HROW, ugr[...][:, D:], ugr[...][:, :D])
    ig = jnp.where(it[...] >= HROW, igr[...][:, D:], igr[...][:, :D])
    g = ug * ig
    out[...] = (jnp.sum(g * wpg[...], axis=1, keepdims=True)
                + jnp.sum(m * wpm[...], axis=1, keepdims=True) + bp[...])


def _tc_mlp(u, it, ugr, igr, um, im, w1a, w1b, b1, w2t, b2, w3t, b3,
            wpg, wpm, bp):
    full = lambda shape: pl.BlockSpec(shape, lambda i: (0,) * len(shape))
    return pl.pallas_call(
        _mlp_body,
        grid=(B // BT,),
        in_specs=[
            pl.BlockSpec((BT, 1), lambda i: (i, 0)),
            pl.BlockSpec((BT, 1), lambda i: (i, 0)),
            pl.BlockSpec((BT, 2 * D), lambda i: (i, 0)),
            pl.BlockSpec((BT, 2 * D), lambda i: (i, 0)),
            pl.BlockSpec((BT, DM), lambda i: (i, 0)),
            pl.BlockSpec((BT, DM), lambda i: (i, 0)),
            full((DM, DM)),
            full((DM, DM)),
            full((1, DM)),
            full((DM, 128)),
            full((1, 128)),
            full((128, D)),
            full((1, D)),
            full((1, D)),
            full((1, D)),
            full((1, 1)),
        ],
        out_specs=pl.BlockSpec((BT, 1), lambda i: (i, 0)),
        out_shape=jax.ShapeDtypeStruct((B, 1), jnp.float32),
    )(u, it, ugr, igr, um, im, w1a, w1b, b1, w2t, b2, w3t, b3, wpg, wpm, bp)


def kernel(user, item, rating, embed_user_GMF, embed_item_GMF,
           embed_user_MLP, embed_item_MLP, W1, b1, W2, b2, W3, b3, Wp, bp):
    user = user.astype(jnp.int32)
    item = item.astype(jnp.int32)
    ug_t = _tc_repack(embed_user_GMF)
    ig_t = _tc_repack(embed_item_GMF)
    um, im = _get_sc_gather_mlp()(user, item, embed_user_MLP, embed_item_MLP)
    tick = (um[0, 0] * 0.0).astype(jnp.int32)
    ugr, igr = _get_sc_gather_gmf()(user % HROW + tick, item % HROW + tick,
                                    ug_t, ig_t)
    w1t = W1.T
    w1a = w1t[:DM]
    w1b = w1t[DM:]
    out = _tc_mlp(user.reshape(B, 1), item.reshape(B, 1), ugr, igr, um, im,
                  w1a, w1b, b1.reshape(1, -1), W2.T, b2.reshape(1, -1),
                  W3.T, b3.reshape(1, -1), Wp[:, :D].reshape(1, D),
                  Wp[:, D:].reshape(1, D), bp.reshape(1, 1))
    return (out, rating)


# speedup vs baseline: 1.2741x; 1.2741x over previous
import functools

import jax
import jax.numpy as jnp
from jax import lax
from jax.experimental import pallas as pl
from jax.experimental.pallas import tpu as pltpu
from jax.experimental.pallas import tpu_sc as plsc

B = 16384
D = 64
DM = 256
NROW = 100000

NC = 2
NS = 16
NW = NC * NS
BPW = B // NW
CH = 128
NCHUNK = BPW // CH
CHG = 256
NCHUNKG = BPW // CHG


def _sc_mesh():
    return plsc.VectorSubcoreMesh(core_axis_name="c", subcore_axis_name="s",
                                  num_cores=NC, num_subcores=NS)


@functools.lru_cache(maxsize=None)
def _get_sc_gather_mlp():
    @functools.partial(
        pl.kernel,
        mesh=_sc_mesh(),
        out_type=[
            jax.ShapeDtypeStruct((B, DM), jnp.float32),
            jax.ShapeDtypeStruct((B, DM), jnp.float32),
        ],
        scratch_types=[
            pltpu.VMEM((CH,), jnp.int32),
            pltpu.VMEM((CH,), jnp.int32),
            pltpu.VMEM((CH, DM), jnp.float32),
            pltpu.VMEM((CH, DM), jnp.float32),
            pltpu.SemaphoreType.DMA,
        ],
    )
    def _sc_gather_mlp(user_hbm, item_hbm, um_t, im_t, um_out, im_out,
                       idxu_v, idxi_v, um_v, im_v, sem):
        wid = lax.axis_index("s") * NC + lax.axis_index("c")
        base = wid * BPW
        for c in range(NCHUNK):
            off = base + c * CH
            pltpu.sync_copy(user_hbm.at[pl.ds(off, CH)], idxu_v)
            pltpu.sync_copy(item_hbm.at[pl.ds(off, CH)], idxi_v)
            cps = [
                pltpu.async_copy(um_t.at[idxu_v], um_v, sem),
                pltpu.async_copy(im_t.at[idxi_v], im_v, sem),
            ]
            for cp in cps:
                cp.wait()
            pltpu.sync_copy(um_v, um_out.at[pl.ds(off, CH)])
            pltpu.sync_copy(im_v, im_out.at[pl.ds(off, CH)])

    return _sc_gather_mlp


@functools.lru_cache(maxsize=None)
def _get_sc_gather_gmf():
    @functools.partial(
        pl.kernel,
        mesh=_sc_mesh(),
        out_type=jax.ShapeDtypeStruct((B, 4 * D), jnp.float32),
        scratch_types=[
            pltpu.VMEM((CHG,), jnp.int32),
            pltpu.VMEM((CHG,), jnp.int32),
            pltpu.VMEM((CHG, 2 * D), jnp.float32),
            pltpu.VMEM((CHG, 2 * D), jnp.float32),
            pltpu.SemaphoreType.DMA,
        ],
    )
    def _sc_gather_gmf(user_hbm, item_hbm, packed_t, g_out,
                       idxu_v, idxi_v, ug_v, ig_v, sem):
        wid = lax.axis_index("s") * NC + lax.axis_index("c")
        base = wid * BPW
        for c in range(NCHUNKG):
            off = base + c * CHG
            pltpu.sync_copy(user_hbm.at[pl.ds(off, CHG)], idxu_v)
            pltpu.sync_copy(item_hbm.at[pl.ds(off, CHG)], idxi_v)
            cps = [
                pltpu.async_copy(packed_t.at[idxu_v], ug_v, sem),
                pltpu.async_copy(packed_t.at[idxi_v], ig_v, sem),
            ]
            for cp in cps:
                cp.wait()
            pltpu.sync_copy(ug_v, g_out.at[pl.ds(off, CHG), pl.ds(0, 2 * D)])
            pltpu.sync_copy(ig_v,
                            g_out.at[pl.ds(off, CHG), pl.ds(2 * D, 2 * D)])

    return _sc_gather_gmf


BT = 2048


def _mlp_body(grow, um, im, w1a, w1b, b1, w2t, b2, w3t, b3,
              wpg, wpm, bp, out):
    h = jnp.dot(um[...], w1a[...], preferred_element_type=jnp.float32)
    h = h + jnp.dot(im[...], w1b[...], preferred_element_type=jnp.float32)
    h = jnp.maximum(h + b1[...], 0.0)
    h = jnp.maximum(jnp.dot(h, w2t[...], preferred_element_type=jnp.float32)
                    + b2[...], 0.0)
    m = jnp.maximum(jnp.dot(h, w3t[...], preferred_element_type=jnp.float32)
                    + b3[...], 0.0)
    g = grow[...][:, :D] * grow[...][:, 3 * D:]
    out[...] = (jnp.sum(g * wpg[...], axis=1, keepdims=True)
                + jnp.sum(m * wpm[...], axis=1, keepdims=True) + bp[...])


def _tc_mlp(grow, um, im, w1a, w1b, b1, w2t, b2, w3t, b3, wpg, wpm, bp):
    full = lambda shape: pl.BlockSpec(shape, lambda i: (0,) * len(shape))
    return pl.pallas_call(
        _mlp_body,
        grid=(B // BT,),
        in_specs=[
            pl.BlockSpec((BT, 4 * D), lambda i: (i, 0)),
            pl.BlockSpec((BT, DM), lambda i: (i, 0)),
            pl.BlockSpec((BT, DM), lambda i: (i, 0)),
            full((DM, DM)),
            full((DM, DM)),
            full((1, DM)),
            full((DM, 128)),
            full((1, 128)),
            full((128, D)),
            full((1, D)),
            full((1, D)),
            full((1, D)),
            full((1, 1)),
        ],
        out_specs=pl.BlockSpec((BT, 1), lambda i: (i, 0)),
        out_shape=jax.ShapeDtypeStruct((B, 1), jnp.float32),
    )(grow, um, im, w1a, w1b, b1, w2t, b2, w3t, b3, wpg, wpm, bp)


def kernel(user, item, rating, embed_user_GMF, embed_item_GMF,
           embed_user_MLP, embed_item_MLP, W1, b1, W2, b2, W3, b3, Wp, bp):
    user = user.astype(jnp.int32)
    item = item.astype(jnp.int32)
    packed = jnp.concatenate([embed_user_GMF, embed_item_GMF], axis=1)
    um, im = _get_sc_gather_mlp()(user, item, embed_user_MLP, embed_item_MLP)
    tick = (um[0, 0] * 0.0).astype(jnp.int32)
    grow = _get_sc_gather_gmf()(user + tick, item + tick, packed)
    w1t = W1.T
    w1a = w1t[:DM]
    w1b = w1t[DM:]
    out = _tc_mlp(grow, um, im, w1a, w1b, b1.reshape(1, -1), W2.T,
                  b2.reshape(1, -1), W3.T, b3.reshape(1, -1),
                  Wp[:, :D].reshape(1, D), Wp[:, D:].reshape(1, D),
                  bp.reshape(1, 1))
    return (out, rating)
